# Initial kernel scaffold; baseline (speedup 1.0000x reference)
import jax
import jax.numpy as jnp
from jax.experimental import pallas as pl


def kernel(nfeats, efeats, edge_index, W1m, b1m, A1, W1a, b1a, W2m, b2m, A2, W2a, b2a, Wc, bc, Wf, bf):
    raise NotImplementedError("write your pallas kernel here")


# speedup vs baseline: 3.5156x; 3.5156x over previous
import functools

import jax
import jax.numpy as jnp
from jax import lax
from jax.experimental import pallas as pl
from jax.experimental.pallas import tpu as pltpu
from jax.experimental.pallas import tpu_sc as plsc

_N = 10000
_E = 320000
_D = 128
_ED = 16

_NC = 2
_NS = 16
_NW = _NC * _NS
_CH = 128
_CHUNKS = 79
_EPW = _CH * _CHUNKS
_EPAD = _NW * _EPW
_NPAD = 10240
_RPT = _NPAD // _NS

_mesh = plsc.VectorSubcoreMesh(core_axis_name="c", subcore_axis_name="s")


def _seg_sum_body(feat_hbm, src_hbm, dst_hbm, zeros_hbm, out_hbm,
                  src_idx, dst_idx, rows, sem, acc_sh, *, indirect):
    c = lax.axis_index("c")
    s = lax.axis_index("s")
    wid = c * _NS + s
    base = wid * _EPW

    pltpu.sync_copy(zeros_hbm, rows)
    for k in range(_RPT // _CH):
        pltpu.sync_copy(rows, acc_sh.at[pl.ds(s * _RPT + k * _CH, _CH)])
    plsc.subcore_barrier()

    def body(i, carry):
        off = base + i * _CH
        pltpu.sync_copy(dst_hbm.at[pl.ds(off, _CH)], dst_idx)
        if indirect:
            pltpu.sync_copy(src_hbm.at[pl.ds(off, _CH)], src_idx)
            pltpu.async_copy(feat_hbm.at[src_idx], rows, sem).wait()
        else:
            pltpu.sync_copy(feat_hbm.at[pl.ds(off, _CH)], rows)
        pltpu.sync_copy(rows, acc_sh.at[dst_idx], add=True)
        return carry

    lax.fori_loop(0, _CHUNKS, body, 0)
    plsc.subcore_barrier()

    for k in range(_RPT // _CH):
        r0 = s * _RPT + k * _CH
        pltpu.sync_copy(acc_sh.at[pl.ds(r0, _CH)], rows)
        pltpu.sync_copy(rows, out_hbm.at[c, pl.ds(r0, _CH)])


def _make_seg_sum(width, indirect):
    return pl.kernel(
        functools.partial(_seg_sum_body, indirect=indirect),
        out_type=jax.ShapeDtypeStruct((_NC, _NPAD, width), jnp.float32),
        mesh=_mesh,
        compiler_params=pltpu.CompilerParams(use_tc_tiling_on_sc=(width == _D)),
        scratch_types=[
            pltpu.VMEM((_CH,), jnp.int32),
            pltpu.VMEM((_CH,), jnp.int32),
            pltpu.VMEM((_CH, width), jnp.float32),
            pltpu.SemaphoreType.DMA,
            pltpu.VMEM_SHARED((_NPAD, width), jnp.float32),
        ],
    )


_seg_sum_h = _make_seg_sum(_D, True)
_seg_sum_ef = _make_seg_sum(32, False)


def _heads_body(psrc_hbm, pdst_hbm, src_hbm, dst_hbm, out_hbm,
                src_idx, dst_idx, acc, sem):
    c = lax.axis_index("c")
    s = lax.axis_index("s")
    wid = c * _NS + s
    base = wid * _EPW

    def body(i, carry):
        off = base + i * _CH
        pltpu.sync_copy(src_hbm.at[pl.ds(off, _CH)], src_idx)
        pltpu.sync_copy(dst_hbm.at[pl.ds(off, _CH)], dst_idx)
        pltpu.async_copy(psrc_hbm.at[src_idx], acc, sem).wait()
        pltpu.async_copy(pdst_hbm.at[dst_idx], acc, sem, add=True).wait()
        pltpu.sync_copy(acc, out_hbm.at[pl.ds(off, _CH)])
        return carry

    lax.fori_loop(0, _CHUNKS, body, 0)


_heads = pl.kernel(
    _heads_body,
    out_type=jax.ShapeDtypeStruct((_EPAD, 16), jnp.float32),
    mesh=_mesh,
    compiler_params=pltpu.CompilerParams(use_tc_tiling_on_sc=False),
    scratch_types=[
        pltpu.VMEM((_CH,), jnp.int32),
        pltpu.VMEM((_CH,), jnp.int32),
        pltpu.VMEM((_CH, 16), jnp.float32),
        pltpu.SemaphoreType.DMA,
    ],
)


def _dense_body(h_ref, s0_ref, s1_ref, t0_ref, t1_ref,
                wm1_ref, wmaug_ref, wa1_ref, wa2_ref, ba_ref, out_ref):
    f32 = jnp.float32
    S = s0_ref[...] + s1_ref[...]
    T = t0_ref[...] + t1_ref[...]
    hn = jnp.dot(S, wm1_ref[...], preferred_element_type=f32)
    hn = hn + jnp.dot(T, wmaug_ref[...], preferred_element_type=f32)
    acc = jnp.dot(h_ref[...], wa1_ref[...], preferred_element_type=f32)
    acc = acc + jnp.dot(hn, wa2_ref[...], preferred_element_type=f32)
    acc = acc + ba_ref[...]
    out_ref[...] = jnp.where(acc >= 0, acc, 0.01 * acc)


_BR = 512


def _tc_dense(h, S0, S1, T0, T1, Wm1, WmAug, Wa1, Wa2, ba_row):
    full = lambda shape: pl.BlockSpec(shape, lambda i: (0, 0))
    row = lambda w: pl.BlockSpec((_BR, w), lambda i: (i, 0))
    return pl.pallas_call(
        _dense_body,
        grid=(_NPAD // _BR,),
        in_specs=[row(_D), row(_D), row(_D), row(32), row(32),
                  full((_D, _D)), full((32, _D)), full((_D, _D)),
                  full((_D, _D)), full((1, _D))],
        out_specs=row(_D),
        out_shape=jax.ShapeDtypeStruct((_NPAD, _D), jnp.float32),
    )(h, S0, S1, T0, T1, Wm1, WmAug, Wa1, Wa2, ba_row)


def _proj_body(h_ref, wsrc_ref, wdst_ref, bias_ref, psrc_ref, pdst_ref):
    f32 = jnp.float32
    h = h_ref[...]
    psrc_ref[...] = jnp.dot(h, wsrc_ref[...], preferred_element_type=f32)
    pdst_ref[...] = (jnp.dot(h, wdst_ref[...], preferred_element_type=f32)
                     + bias_ref[...])


def _tc_proj(h, Wsrc, Wdst, bias_row):
    full = lambda shape: pl.BlockSpec(shape, lambda i: (0, 0))
    row = lambda w: pl.BlockSpec((_BR, w), lambda i: (i, 0))
    return pl.pallas_call(
        _proj_body,
        grid=(_NPAD // _BR,),
        in_specs=[row(_D), full((_D, 16)), full((_D, 16)), full((1, 16))],
        out_specs=[row(16), row(16)],
        out_shape=[jax.ShapeDtypeStruct((_NPAD, 16), jnp.float32),
                   jax.ShapeDtypeStruct((_NPAD, 16), jnp.float32)],
    )(h, Wsrc, Wdst, bias_row)


def kernel(nfeats, efeats, edge_index, W1m, b1m, A1, W1a, b1a,
           W2m, b2m, A2, W2a, b2a, Wc, bc, Wf, bf):
    f32 = jnp.float32
    h0 = nfeats[:, 0, :]
    ef = efeats[:, 0, :]
    src = edge_index[0]
    dst = edge_index[1]

    pad_e = _EPAD - _E
    srcp = jnp.concatenate([src, jnp.zeros((pad_e,), jnp.int32)])
    dstp = jnp.concatenate([dst, jnp.full((pad_e,), _N, jnp.int32)])
    ef32 = jnp.concatenate(
        [ef, jnp.ones((_E, 1), f32), jnp.zeros((_E, 15), f32)], axis=1)
    ef32 = jnp.concatenate([ef32, jnp.zeros((pad_e, 32), f32)], axis=0)
    h0p = jnp.concatenate([h0, jnp.zeros((_NPAD - _N, _D), f32)], axis=0)
    z128 = jnp.zeros((_CH, _D), f32)
    z32 = jnp.zeros((_CH, 32), f32)

    WmAug1 = jnp.concatenate([W1m[_D:], b1m[None], jnp.zeros((15, _D), f32)])
    WmAug2 = jnp.concatenate([W2m[_D:], b2m[None], jnp.zeros((15, _D), f32)])

    Taug = _seg_sum_ef(ef32, srcp, dstp, z32)
    S1 = _seg_sum_h(h0p, srcp, dstp, z128)
    h1 = _tc_dense(h0p, S1[0], S1[1], Taug[0], Taug[1],
                   W1m[:_D], WmAug1, W1a[:_D], W1a[_D:], b1a[None])
    S2 = _seg_sum_h(h1, srcp, dstp, z128)
    h2 = _tc_dense(h1, S2[0], S2[1], Taug[0], Taug[1],
                   W2m[:_D], WmAug2, W2a[:_D], W2a[_D:], b2a[None])

    Wsrc = jnp.concatenate([Wc[:_D], Wf[:_D], jnp.zeros((_D, 4), f32)], axis=1)
    Wdst = jnp.concatenate([Wc[_D:], Wf[_D:], jnp.zeros((_D, 4), f32)], axis=1)
    bias16 = jnp.concatenate([bc, bf, jnp.zeros((4,), f32)])[None]
    Psrc, Pdst = _tc_proj(h2, Wsrc, Wdst, bias16)

    out16 = _heads(Psrc, Pdst, srcp, dstp)
    coarse = out16[:_E, 0:2]
    fine = out16[:_E, 2:12]
    return coarse, fine
